# baseline (speedup 1.0000x reference)
import functools

import jax
import jax.numpy as jnp
from jax import lax
from jax.experimental import pallas as pl
from jax.experimental.pallas import tpu as pltpu
from jax.experimental.pallas import tpu_sc as plsc

N = 10000
E = 320000
D = 128
DH = D // 2
NC = 2
NS = 16
NW = NC * NS
EPT = E // NW
CH = 80
NCHUNK = 250
NBUF = 6
STRIPE = 1000
ZROWS = 200
RB = 1000
GRID = N // RB

_mesh = plsc.VectorSubcoreMesh(
    core_axis_name="c", subcore_axis_name="s", num_cores=NC, num_subcores=NS)
_sc_params = pltpu.CompilerParams(needs_layout_passes=False,
                                  use_tc_tiling_on_sc=False)


@functools.partial(
    pl.kernel,
    out_type=[jax.ShapeDtypeStruct((GRID, NW, RB), jnp.float32),
              jax.ShapeDtypeStruct((GRID, NW, RB), jnp.float32)],
    mesh=_mesh,
    scratch_types=[pltpu.VMEM((EPT,), jnp.int32),
                   pltpu.VMEM((EPT,), jnp.int32),
                   pltpu.VMEM((N,), jnp.float32),
                   pltpu.VMEM((N,), jnp.float32)],
    compiler_params=_sc_params,
)
def _deg_kernel(e_hbm, out_src, out_dst, src_v, dst_v, hs, hd):
    c = lax.axis_index("c")
    s = lax.axis_index("s")
    w = c * NS + s
    pltpu.sync_copy(e_hbm.at[0, w], src_v)
    pltpu.sync_copy(e_hbm.at[1, w], dst_v)

    zeros = jnp.zeros((16,), jnp.float32)

    def zbody(i, carry):
        hs[pl.ds(i * 16, 16)] = zeros
        hd[pl.ds(i * 16, 16)] = zeros
        return carry

    lax.fori_loop(0, N // 16, zbody, 0)

    ones = jnp.ones((16,), jnp.float32)

    def hbody(i, carry):
        plsc.addupdate_scatter(hs, [src_v[pl.ds(i * 16, 16)]], ones)
        plsc.addupdate_scatter(hd, [dst_v[pl.ds(i * 16, 16)]], ones)
        return carry

    lax.fori_loop(0, EPT // 16, hbody, 0)

    for g in range(GRID):
        pltpu.sync_copy(hs.at[pl.ds(g * RB, RB)], out_src.at[g, w])
        pltpu.sync_copy(hd.at[pl.ds(g * RB, RB)], out_dst.at[g, w])


@functools.partial(
    pl.kernel,
    out_type=jax.ShapeDtypeStruct((NC, N, DH), jnp.float32),
    mesh=_mesh,
    scratch_types=[pltpu.VMEM((NCHUNK, CH), jnp.int32),
                   pltpu.VMEM((NCHUNK, CH), jnp.int32),
                   pltpu.VMEM((CH, DH), jnp.float32),
                   pltpu.VMEM((CH, DH), jnp.float32),
                   pltpu.VMEM((CH, DH), jnp.float32),
                   pltpu.VMEM((CH, DH), jnp.float32),
                   pltpu.VMEM((CH, DH), jnp.float32),
                   pltpu.VMEM((CH, DH), jnp.float32),
                   pltpu.VMEM_SHARED((N, DH), jnp.float32),
                   pltpu.SemaphoreType.DMA,
                   pltpu.SemaphoreType.DMA,
                   pltpu.SemaphoreType.DMA,
                   pltpu.SemaphoreType.DMA,
                   pltpu.SemaphoreType.DMA,
                   pltpu.SemaphoreType.DMA],
    compiler_params=_sc_params,
)
def _agg_kernel(h_hbm, eg_hbm, out_hbm,
                src_v, dst_v, rows_a, rows_b, rows_c, rows_d, rows_e, rows_f,
                agg_sh, sem_a, sem_b, sem_c, sem_d, sem_e, sem_f):
    c = lax.axis_index("c")
    s = lax.axis_index("s")

    zeros = jnp.zeros((16,), jnp.float32)

    def zfill(i, carry):
        for k in range(DH // 16):
            rows_a[i, pl.ds(k * 16, 16)] = zeros
        return carry

    lax.fori_loop(0, CH, zfill, 0)

    @pl.when(s < N // STRIPE)
    def _zero():
        for t in range(STRIPE // CH):
            pltpu.sync_copy(rows_a,
                            agg_sh.at[pl.ds(s * STRIPE + t * CH, CH)])
        pltpu.sync_copy(rows_a.at[pl.ds(0, STRIPE % CH)],
                        agg_sh.at[pl.ds(s * STRIPE + STRIPE - STRIPE % CH,
                                        STRIPE % CH)])

    plsc.subcore_barrier()

    pltpu.sync_copy(eg_hbm.at[0, s], src_v)
    pltpu.sync_copy(eg_hbm.at[1, s], dst_v)

    h_half = h_hbm.at[c]
    bufs = (rows_a, rows_b, rows_c, rows_d, rows_e, rows_f)
    sems = (sem_a, sem_b, sem_c, sem_d, sem_e, sem_f)
    depth = NBUF - 1

    def fire_gather(j, u):
        pltpu.async_copy(h_half.at[src_v.at[j]], bufs[u], sems[u])

    def wait_gather(j, u):
        pltpu.make_async_copy(h_half.at[src_v.at[j]], bufs[u],
                              sems[u]).wait()

    for j in range(depth):
        fire_gather(j, j)

    NMAIN = (NCHUNK - 2 * depth) // NBUF

    def body(k, carry):
        j0 = k * NBUF
        for u in range(NBUF):
            j = j0 + u
            wait_gather(j, u)
            fire_gather(j + depth, (u + depth) % NBUF)
            pltpu.sync_copy(bufs[u], agg_sh.at[dst_v.at[j]], add=True)
        return carry

    lax.fori_loop(0, NMAIN, body, 0)

    for j in range(NMAIN * NBUF, NCHUNK):
        u = j % NBUF
        wait_gather(j, u)
        if j + depth < NCHUNK:
            fire_gather(j + depth, (u + depth) % NBUF)
        pltpu.sync_copy(bufs[u], agg_sh.at[dst_v.at[j]], add=True)
    plsc.subcore_barrier()

    @pl.when(s < N // STRIPE)
    def _copy_out():
        pltpu.sync_copy(agg_sh.at[pl.ds(s * STRIPE, STRIPE)],
                        out_hbm.at[c, pl.ds(s * STRIPE, STRIPE)])


def _mm1_body(deg_ref, x_ref, w_ref, h_ref):
    deg = jnp.sum(deg_ref[0], axis=0)
    nsrc = lax.rsqrt(jnp.maximum(deg, 1.0))
    xb = x_ref[...] * nsrc[:, None]
    res = jnp.dot(xb, w_ref[...],
                  preferred_element_type=jnp.float32,
                  precision=lax.Precision.HIGHEST)
    h_ref[0] = res[:, :DH]
    h_ref[1] = res[:, DH:]


def _mm1(deg_parts, x, w_conv):
    return pl.pallas_call(
        _mm1_body,
        grid=(GRID,),
        in_specs=[
            pl.BlockSpec((1, NW, RB), lambda i: (i, 0, 0)),
            pl.BlockSpec((RB, D), lambda i: (i, 0)),
            pl.BlockSpec((D, D), lambda i: (0, 0)),
        ],
        out_specs=pl.BlockSpec((NC, RB, DH), lambda i: (0, i, 0)),
        out_shape=jax.ShapeDtypeStruct((NC, N, DH), jnp.float32),
    )(deg_parts, x, w_conv)


def _mlp_body(agg_ref, deg_ref, bc_ref, wf_ref, bf_ref, wf2_ref, bf2_ref,
              out_ref):
    agg = jnp.concatenate([agg_ref[0], agg_ref[1]], axis=-1)
    deg = jnp.sum(deg_ref[0], axis=0)
    ndst = lax.rsqrt(jnp.maximum(deg, 1.0))
    h = agg * ndst[:, None] + bc_ref[...]
    h = jnp.maximum(h, 0.0)
    h = jnp.dot(h, wf_ref[...], preferred_element_type=jnp.float32,
                precision=lax.Precision.HIGHEST) + bf_ref[...]
    h = jnp.maximum(h, 0.0)
    out_ref[...] = jnp.dot(h, wf2_ref[...], preferred_element_type=jnp.float32,
                           precision=lax.Precision.HIGHEST) + bf2_ref[...]


def _mlp(agg_parts, deg_parts, b_conv, w_fc, b_fc, w_fc2, b_fc2):
    return pl.pallas_call(
        _mlp_body,
        grid=(GRID,),
        in_specs=[
            pl.BlockSpec((NC, RB, DH), lambda i: (0, i, 0)),
            pl.BlockSpec((1, NW, RB), lambda i: (i, 0, 0)),
            pl.BlockSpec((1, D), lambda i: (0, 0)),
            pl.BlockSpec((D, D), lambda i: (0, 0)),
            pl.BlockSpec((1, D), lambda i: (0, 0)),
            pl.BlockSpec((D, D), lambda i: (0, 0)),
            pl.BlockSpec((1, D), lambda i: (0, 0)),
        ],
        out_specs=pl.BlockSpec((RB, D), lambda i: (i, 0)),
        out_shape=jax.ShapeDtypeStruct((N, D), jnp.float32),
    )(agg_parts, deg_parts, b_conv, w_fc, b_fc, w_fc2, b_fc2)


def kernel(x, edge_index, W_conv, b_conv, W_fc, b_fc, W_fc2, b_fc2):
    ei = edge_index.astype(jnp.int32)
    eh = ei.reshape(2, NW, EPT)
    eg = ei.reshape(2, NS, NCHUNK, CH)

    deg_out_parts, deg_in_parts = _deg_kernel(eh)
    h = _mm1(deg_out_parts, x, W_conv)
    agg_parts = _agg_kernel(h, eg)
    return _mlp(agg_parts, deg_in_parts, b_conv.reshape(1, D),
                W_fc, b_fc.reshape(1, D), W_fc2, b_fc2.reshape(1, D))


# speedup vs baseline: 1.0198x; 1.0198x over previous
import functools

import jax
import jax.numpy as jnp
from jax import lax
from jax.experimental import pallas as pl
from jax.experimental.pallas import tpu as pltpu
from jax.experimental.pallas import tpu_sc as plsc

N = 10000
E = 320000
D = 128
DH = D // 2
NC = 2
NS = 16
NW = NC * NS
EPT = E // NW
CH = 80
NCHUNK = 250
NBUF = 6
STRIPE = 1000
ZROWS = 200
RB = 5000
GRID = N // RB

_mesh = plsc.VectorSubcoreMesh(
    core_axis_name="c", subcore_axis_name="s", num_cores=NC, num_subcores=NS)
_sc_params = pltpu.CompilerParams(needs_layout_passes=False,
                                  use_tc_tiling_on_sc=False)


@functools.partial(
    pl.kernel,
    out_type=[jax.ShapeDtypeStruct((GRID, NW, RB), jnp.float32),
              jax.ShapeDtypeStruct((GRID, NW, RB), jnp.float32)],
    mesh=_mesh,
    scratch_types=[pltpu.VMEM((EPT,), jnp.int32),
                   pltpu.VMEM((EPT,), jnp.int32),
                   pltpu.VMEM((N,), jnp.float32),
                   pltpu.VMEM((N,), jnp.float32)],
    compiler_params=_sc_params,
)
def _deg_kernel(e_hbm, out_src, out_dst, src_v, dst_v, hs, hd):
    c = lax.axis_index("c")
    s = lax.axis_index("s")
    w = c * NS + s
    pltpu.sync_copy(e_hbm.at[0, w], src_v)
    pltpu.sync_copy(e_hbm.at[1, w], dst_v)

    zeros = jnp.zeros((16,), jnp.float32)

    def zbody(i, carry):
        hs[pl.ds(i * 16, 16)] = zeros
        hd[pl.ds(i * 16, 16)] = zeros
        return carry

    lax.fori_loop(0, N // 16, zbody, 0)

    ones = jnp.ones((16,), jnp.float32)

    def hbody(i, carry):
        plsc.addupdate_scatter(hs, [src_v[pl.ds(i * 16, 16)]], ones)
        plsc.addupdate_scatter(hd, [dst_v[pl.ds(i * 16, 16)]], ones)
        return carry

    lax.fori_loop(0, EPT // 16, hbody, 0)

    for g in range(GRID):
        pltpu.sync_copy(hs.at[pl.ds(g * RB, RB)], out_src.at[g, w])
        pltpu.sync_copy(hd.at[pl.ds(g * RB, RB)], out_dst.at[g, w])


@functools.partial(
    pl.kernel,
    out_type=jax.ShapeDtypeStruct((NC, N, DH), jnp.float32),
    mesh=_mesh,
    scratch_types=[pltpu.VMEM((NCHUNK, CH), jnp.int32),
                   pltpu.VMEM((NCHUNK, CH), jnp.int32),
                   pltpu.VMEM((CH, DH), jnp.float32),
                   pltpu.VMEM((CH, DH), jnp.float32),
                   pltpu.VMEM((CH, DH), jnp.float32),
                   pltpu.VMEM((CH, DH), jnp.float32),
                   pltpu.VMEM((CH, DH), jnp.float32),
                   pltpu.VMEM((CH, DH), jnp.float32),
                   pltpu.VMEM_SHARED((N, DH), jnp.float32),
                   pltpu.SemaphoreType.DMA,
                   pltpu.SemaphoreType.DMA,
                   pltpu.SemaphoreType.DMA,
                   pltpu.SemaphoreType.DMA,
                   pltpu.SemaphoreType.DMA,
                   pltpu.SemaphoreType.DMA],
    compiler_params=_sc_params,
)
def _agg_kernel(h_hbm, eg_hbm, out_hbm,
                src_v, dst_v, rows_a, rows_b, rows_c, rows_d, rows_e, rows_f,
                agg_sh, sem_a, sem_b, sem_c, sem_d, sem_e, sem_f):
    c = lax.axis_index("c")
    s = lax.axis_index("s")

    zeros = jnp.zeros((16,), jnp.float32)

    def zfill(i, carry):
        for k in range(DH // 16):
            rows_a[i, pl.ds(k * 16, 16)] = zeros
        return carry

    lax.fori_loop(0, CH, zfill, 0)

    @pl.when(s < N // STRIPE)
    def _zero():
        for t in range(STRIPE // CH):
            pltpu.sync_copy(rows_a,
                            agg_sh.at[pl.ds(s * STRIPE + t * CH, CH)])
        pltpu.sync_copy(rows_a.at[pl.ds(0, STRIPE % CH)],
                        agg_sh.at[pl.ds(s * STRIPE + STRIPE - STRIPE % CH,
                                        STRIPE % CH)])

    plsc.subcore_barrier()

    pltpu.sync_copy(eg_hbm.at[0, s], src_v)
    pltpu.sync_copy(eg_hbm.at[1, s], dst_v)

    h_half = h_hbm.at[c]
    bufs = (rows_a, rows_b, rows_c, rows_d, rows_e, rows_f)
    sems = (sem_a, sem_b, sem_c, sem_d, sem_e, sem_f)
    depth = NBUF - 1

    def fire_gather(j, u):
        pltpu.async_copy(h_half.at[src_v.at[j]], bufs[u], sems[u])

    def wait_gather(j, u):
        pltpu.make_async_copy(h_half.at[src_v.at[j]], bufs[u],
                              sems[u]).wait()

    for j in range(depth):
        fire_gather(j, j)

    NMAIN = (NCHUNK - 2 * depth) // NBUF

    def body(k, carry):
        j0 = k * NBUF
        for u in range(NBUF):
            j = j0 + u
            wait_gather(j, u)
            fire_gather(j + depth, (u + depth) % NBUF)
            pltpu.sync_copy(bufs[u], agg_sh.at[dst_v.at[j]], add=True)
        return carry

    lax.fori_loop(0, NMAIN, body, 0)

    for j in range(NMAIN * NBUF, NCHUNK):
        u = j % NBUF
        wait_gather(j, u)
        if j + depth < NCHUNK:
            fire_gather(j + depth, (u + depth) % NBUF)
        pltpu.sync_copy(bufs[u], agg_sh.at[dst_v.at[j]], add=True)
    plsc.subcore_barrier()

    @pl.when(s < N // STRIPE)
    def _copy_out():
        pltpu.sync_copy(agg_sh.at[pl.ds(s * STRIPE, STRIPE)],
                        out_hbm.at[c, pl.ds(s * STRIPE, STRIPE)])


def _mm1_body(deg_ref, x_ref, w_ref, h_ref):
    deg = jnp.sum(deg_ref[0], axis=0)
    nsrc = lax.rsqrt(jnp.maximum(deg, 1.0))
    xb = x_ref[...] * nsrc[:, None]
    res = jnp.dot(xb, w_ref[...],
                  preferred_element_type=jnp.float32,
                  precision=lax.Precision.HIGHEST)
    h_ref[0] = res[:, :DH]
    h_ref[1] = res[:, DH:]


def _mm1(deg_parts, x, w_conv):
    return pl.pallas_call(
        _mm1_body,
        grid=(GRID,),
        in_specs=[
            pl.BlockSpec((1, NW, RB), lambda i: (i, 0, 0)),
            pl.BlockSpec((RB, D), lambda i: (i, 0)),
            pl.BlockSpec((D, D), lambda i: (0, 0)),
        ],
        out_specs=pl.BlockSpec((NC, RB, DH), lambda i: (0, i, 0)),
        out_shape=jax.ShapeDtypeStruct((NC, N, DH), jnp.float32),
    )(deg_parts, x, w_conv)


def _mlp_body(agg_ref, deg_ref, bc_ref, wf_ref, bf_ref, wf2_ref, bf2_ref,
              out_ref):
    agg = jnp.concatenate([agg_ref[0], agg_ref[1]], axis=-1)
    deg = jnp.sum(deg_ref[0], axis=0)
    ndst = lax.rsqrt(jnp.maximum(deg, 1.0))
    h = agg * ndst[:, None] + bc_ref[...]
    h = jnp.maximum(h, 0.0)
    h = jnp.dot(h, wf_ref[...], preferred_element_type=jnp.float32,
                precision=lax.Precision.HIGHEST) + bf_ref[...]
    h = jnp.maximum(h, 0.0)
    out_ref[...] = jnp.dot(h, wf2_ref[...], preferred_element_type=jnp.float32,
                           precision=lax.Precision.HIGHEST) + bf2_ref[...]


def _mlp(agg_parts, deg_parts, b_conv, w_fc, b_fc, w_fc2, b_fc2):
    return pl.pallas_call(
        _mlp_body,
        grid=(GRID,),
        in_specs=[
            pl.BlockSpec((NC, RB, DH), lambda i: (0, i, 0)),
            pl.BlockSpec((1, NW, RB), lambda i: (i, 0, 0)),
            pl.BlockSpec((1, D), lambda i: (0, 0)),
            pl.BlockSpec((D, D), lambda i: (0, 0)),
            pl.BlockSpec((1, D), lambda i: (0, 0)),
            pl.BlockSpec((D, D), lambda i: (0, 0)),
            pl.BlockSpec((1, D), lambda i: (0, 0)),
        ],
        out_specs=pl.BlockSpec((RB, D), lambda i: (i, 0)),
        out_shape=jax.ShapeDtypeStruct((N, D), jnp.float32),
    )(agg_parts, deg_parts, b_conv, w_fc, b_fc, w_fc2, b_fc2)


def kernel(x, edge_index, W_conv, b_conv, W_fc, b_fc, W_fc2, b_fc2):
    ei = edge_index.astype(jnp.int32)
    eh = ei.reshape(2, NW, EPT)
    eg = ei.reshape(2, NS, NCHUNK, CH)

    deg_out_parts, deg_in_parts = _deg_kernel(eh)
    h = _mm1(deg_out_parts, x, W_conv)
    agg_parts = _agg_kernel(h, eg)
    return _mlp(agg_parts, deg_in_parts, b_conv.reshape(1, D),
                W_fc, b_fc.reshape(1, D), W_fc2, b_fc2.reshape(1, D))


# speedup vs baseline: 1.1916x; 1.1684x over previous
import functools

import jax
import jax.numpy as jnp
from jax import lax
from jax.experimental import pallas as pl
from jax.experimental.pallas import tpu as pltpu
from jax.experimental.pallas import tpu_sc as plsc

N = 10000
E = 320000
D = 128
DH = D // 2
NC = 2
NS = 16
NW = NC * NS
EPT = E // NW
CH = 80
NCHUNK = 250
NBUF = 6
STRIPE = 1000
ZROWS = 200
RB = 2000
GRID = N // RB

_mesh = plsc.VectorSubcoreMesh(
    core_axis_name="c", subcore_axis_name="s", num_cores=NC, num_subcores=NS)
_sc_params = pltpu.CompilerParams(needs_layout_passes=False,
                                  use_tc_tiling_on_sc=False)


@functools.partial(
    pl.kernel,
    out_type=[jax.ShapeDtypeStruct((GRID, NW, RB), jnp.float32),
              jax.ShapeDtypeStruct((GRID, NW, RB), jnp.float32)],
    mesh=_mesh,
    scratch_types=[pltpu.VMEM((EPT,), jnp.int32),
                   pltpu.VMEM((EPT,), jnp.int32),
                   pltpu.VMEM((N,), jnp.float32),
                   pltpu.VMEM((N,), jnp.float32)],
    compiler_params=_sc_params,
)
def _deg_kernel(e_hbm, out_src, out_dst, src_v, dst_v, hs, hd):
    c = lax.axis_index("c")
    s = lax.axis_index("s")
    w = c * NS + s
    pltpu.sync_copy(e_hbm.at[0, w], src_v)
    pltpu.sync_copy(e_hbm.at[1, w], dst_v)

    zeros = jnp.zeros((16,), jnp.float32)

    def zbody(i, carry):
        hs[pl.ds(i * 16, 16)] = zeros
        hd[pl.ds(i * 16, 16)] = zeros
        return carry

    lax.fori_loop(0, N // 16, zbody, 0)

    ones = jnp.ones((16,), jnp.float32)

    def hbody(i, carry):
        plsc.addupdate_scatter(hs, [src_v[pl.ds(i * 16, 16)]], ones)
        plsc.addupdate_scatter(hd, [dst_v[pl.ds(i * 16, 16)]], ones)
        return carry

    lax.fori_loop(0, EPT // 16, hbody, 0)

    for g in range(GRID):
        pltpu.sync_copy(hs.at[pl.ds(g * RB, RB)], out_src.at[g, w])
        pltpu.sync_copy(hd.at[pl.ds(g * RB, RB)], out_dst.at[g, w])


@functools.partial(
    pl.kernel,
    out_type=jax.ShapeDtypeStruct((NC, N, DH), jnp.float32),
    mesh=_mesh,
    scratch_types=[pltpu.VMEM((NCHUNK, CH), jnp.int32),
                   pltpu.VMEM((NCHUNK, CH), jnp.int32),
                   pltpu.VMEM((CH, DH), jnp.float32),
                   pltpu.VMEM((CH, DH), jnp.float32),
                   pltpu.VMEM((CH, DH), jnp.float32),
                   pltpu.VMEM((CH, DH), jnp.float32),
                   pltpu.VMEM((CH, DH), jnp.float32),
                   pltpu.VMEM((CH, DH), jnp.float32),
                   pltpu.VMEM_SHARED((N, DH), jnp.float32),
                   pltpu.SemaphoreType.DMA,
                   pltpu.SemaphoreType.DMA,
                   pltpu.SemaphoreType.DMA,
                   pltpu.SemaphoreType.DMA,
                   pltpu.SemaphoreType.DMA,
                   pltpu.SemaphoreType.DMA],
    compiler_params=_sc_params,
)
def _agg_kernel(h_hbm, eg_hbm, out_hbm,
                src_v, dst_v, rows_a, rows_b, rows_c, rows_d, rows_e, rows_f,
                agg_sh, sem_a, sem_b, sem_c, sem_d, sem_e, sem_f):
    c = lax.axis_index("c")
    s = lax.axis_index("s")

    zeros = jnp.zeros((16,), jnp.float32)

    def zfill(i, carry):
        for k in range(DH // 16):
            rows_a[i, pl.ds(k * 16, 16)] = zeros
        return carry

    lax.fori_loop(0, CH, zfill, 0)

    @pl.when(s < N // STRIPE)
    def _zero():
        for t in range(STRIPE // CH):
            pltpu.sync_copy(rows_a,
                            agg_sh.at[pl.ds(s * STRIPE + t * CH, CH)])
        pltpu.sync_copy(rows_a.at[pl.ds(0, STRIPE % CH)],
                        agg_sh.at[pl.ds(s * STRIPE + STRIPE - STRIPE % CH,
                                        STRIPE % CH)])

    plsc.subcore_barrier()

    pltpu.sync_copy(eg_hbm.at[0, s], src_v)
    pltpu.sync_copy(eg_hbm.at[1, s], dst_v)

    h_half = h_hbm.at[c]
    bufs = (rows_a, rows_b, rows_c, rows_d, rows_e, rows_f)
    sems = (sem_a, sem_b, sem_c, sem_d, sem_e, sem_f)
    depth = NBUF - 1

    def fire_gather(j, u):
        pltpu.async_copy(h_half.at[src_v.at[j]], bufs[u], sems[u])

    def wait_gather(j, u):
        pltpu.make_async_copy(h_half.at[src_v.at[j]], bufs[u],
                              sems[u]).wait()

    for j in range(depth):
        fire_gather(j, j)

    NMAIN = (NCHUNK - 2 * depth) // NBUF

    def body(k, carry):
        j0 = k * NBUF
        for u in range(NBUF):
            j = j0 + u
            wait_gather(j, u)
            fire_gather(j + depth, (u + depth) % NBUF)
            pltpu.sync_copy(bufs[u], agg_sh.at[dst_v.at[j]], add=True)
        return carry

    lax.fori_loop(0, NMAIN, body, 0)

    for j in range(NMAIN * NBUF, NCHUNK):
        u = j % NBUF
        wait_gather(j, u)
        if j + depth < NCHUNK:
            fire_gather(j + depth, (u + depth) % NBUF)
        pltpu.sync_copy(bufs[u], agg_sh.at[dst_v.at[j]], add=True)
    plsc.subcore_barrier()

    @pl.when(s < N // STRIPE)
    def _copy_out():
        pltpu.sync_copy(agg_sh.at[pl.ds(s * STRIPE, STRIPE)],
                        out_hbm.at[c, pl.ds(s * STRIPE, STRIPE)])


def _mm1_body(deg_ref, x_ref, w_ref, h_ref):
    deg = jnp.sum(deg_ref[0], axis=0)
    nsrc = lax.rsqrt(jnp.maximum(deg, 1.0))
    xb = x_ref[...] * nsrc[:, None]
    res = jnp.dot(xb, w_ref[...],
                  preferred_element_type=jnp.float32)
    h_ref[0] = res[:, :DH]
    h_ref[1] = res[:, DH:]


def _mm1(deg_parts, x, w_conv):
    return pl.pallas_call(
        _mm1_body,
        grid=(GRID,),
        in_specs=[
            pl.BlockSpec((1, NW, RB), lambda i: (i, 0, 0)),
            pl.BlockSpec((RB, D), lambda i: (i, 0)),
            pl.BlockSpec((D, D), lambda i: (0, 0)),
        ],
        out_specs=pl.BlockSpec((NC, RB, DH), lambda i: (0, i, 0)),
        out_shape=jax.ShapeDtypeStruct((NC, N, DH), jnp.float32),
    )(deg_parts, x, w_conv)


def _mlp_body(agg_ref, deg_ref, bc_ref, wf_ref, bf_ref, wf2_ref, bf2_ref,
              out_ref):
    agg = jnp.concatenate([agg_ref[0], agg_ref[1]], axis=-1)
    deg = jnp.sum(deg_ref[0], axis=0)
    ndst = lax.rsqrt(jnp.maximum(deg, 1.0))
    h = agg * ndst[:, None] + bc_ref[...]
    h = jnp.maximum(h, 0.0)
    h = jnp.dot(h, wf_ref[...],
                preferred_element_type=jnp.float32) + bf_ref[...]
    h = jnp.maximum(h, 0.0)
    out_ref[...] = jnp.dot(h, wf2_ref[...],
                           preferred_element_type=jnp.float32) + bf2_ref[...]


def _mlp(agg_parts, deg_parts, b_conv, w_fc, b_fc, w_fc2, b_fc2):
    return pl.pallas_call(
        _mlp_body,
        grid=(GRID,),
        in_specs=[
            pl.BlockSpec((NC, RB, DH), lambda i: (0, i, 0)),
            pl.BlockSpec((1, NW, RB), lambda i: (i, 0, 0)),
            pl.BlockSpec((1, D), lambda i: (0, 0)),
            pl.BlockSpec((D, D), lambda i: (0, 0)),
            pl.BlockSpec((1, D), lambda i: (0, 0)),
            pl.BlockSpec((D, D), lambda i: (0, 0)),
            pl.BlockSpec((1, D), lambda i: (0, 0)),
        ],
        out_specs=pl.BlockSpec((RB, D), lambda i: (i, 0)),
        out_shape=jax.ShapeDtypeStruct((N, D), jnp.float32),
    )(agg_parts, deg_parts, b_conv, w_fc, b_fc, w_fc2, b_fc2)


def kernel(x, edge_index, W_conv, b_conv, W_fc, b_fc, W_fc2, b_fc2):
    ei = edge_index.astype(jnp.int32)
    eh = ei.reshape(2, NW, EPT)
    eg = ei.reshape(2, NS, NCHUNK, CH)

    deg_out_parts, deg_in_parts = _deg_kernel(eh)
    h = _mm1(deg_out_parts, x, W_conv)
    agg_parts = _agg_kernel(h, eg)
    return _mlp(agg_parts, deg_in_parts, b_conv.reshape(1, D),
                W_fc, b_fc.reshape(1, D), W_fc2, b_fc2.reshape(1, D))


# speedup vs baseline: 1.1929x; 1.0011x over previous
import functools

import jax
import jax.numpy as jnp
from jax import lax
from jax.experimental import pallas as pl
from jax.experimental.pallas import tpu as pltpu
from jax.experimental.pallas import tpu_sc as plsc

N = 10000
E = 320000
D = 128
DH = D // 2
NC = 2
NS = 16
NW = NC * NS
EPT = E // NW
CH = 80
NCHUNK = 250
NBUF = 6
STRIPE = 1000
RB = 2000
GRID = N // RB

_mesh = plsc.VectorSubcoreMesh(
    core_axis_name="c", subcore_axis_name="s", num_cores=NC, num_subcores=NS)
_sc_params = pltpu.CompilerParams(needs_layout_passes=False,
                                  use_tc_tiling_on_sc=False)


@functools.partial(
    pl.kernel,
    out_type=[jax.ShapeDtypeStruct((GRID, NW, RB), jnp.float32),
              jax.ShapeDtypeStruct((GRID, NW, RB), jnp.float32)],
    mesh=_mesh,
    scratch_types=[pltpu.VMEM((EPT,), jnp.int32),
                   pltpu.VMEM((EPT,), jnp.int32),
                   pltpu.VMEM((N,), jnp.float32),
                   pltpu.VMEM((N,), jnp.float32)],
    compiler_params=_sc_params,
)
def _deg_kernel(e_hbm, out_src, out_dst, src_v, dst_v, hs, hd):
    c = lax.axis_index("c")
    s = lax.axis_index("s")
    w = c * NS + s
    pltpu.sync_copy(e_hbm.at[0, w], src_v)
    pltpu.sync_copy(e_hbm.at[1, w], dst_v)

    zeros = jnp.zeros((16,), jnp.float32)

    def zbody(i, carry):
        hs[pl.ds(i * 16, 16)] = zeros
        hd[pl.ds(i * 16, 16)] = zeros
        return carry

    lax.fori_loop(0, N // 16, zbody, 0)

    ones = jnp.ones((16,), jnp.float32)

    def hbody(i, carry):
        plsc.addupdate_scatter(hs, [src_v[pl.ds(i * 16, 16)]], ones)
        plsc.addupdate_scatter(hd, [dst_v[pl.ds(i * 16, 16)]], ones)
        return carry

    lax.fori_loop(0, EPT // 16, hbody, 0)

    for g in range(GRID):
        pltpu.sync_copy(hs.at[pl.ds(g * RB, RB)], out_src.at[g, w])
        pltpu.sync_copy(hd.at[pl.ds(g * RB, RB)], out_dst.at[g, w])


@functools.partial(
    pl.kernel,
    out_type=jax.ShapeDtypeStruct((NC, N, DH), jnp.float32),
    mesh=_mesh,
    scratch_types=[pltpu.VMEM((NCHUNK, CH), jnp.int32),
                   pltpu.VMEM((NCHUNK, CH), jnp.int32),
                   pltpu.VMEM((CH, DH), jnp.float32),
                   pltpu.VMEM((CH, DH), jnp.float32),
                   pltpu.VMEM((CH, DH), jnp.float32),
                   pltpu.VMEM((CH, DH), jnp.float32),
                   pltpu.VMEM((CH, DH), jnp.float32),
                   pltpu.VMEM((CH, DH), jnp.float32),
                   pltpu.VMEM_SHARED((N, DH), jnp.float32),
                   pltpu.SemaphoreType.DMA,
                   pltpu.SemaphoreType.DMA,
                   pltpu.SemaphoreType.DMA,
                   pltpu.SemaphoreType.DMA,
                   pltpu.SemaphoreType.DMA,
                   pltpu.SemaphoreType.DMA],
    compiler_params=_sc_params,
)
def _agg_kernel(h_hbm, eg_hbm, out_hbm,
                src_v, dst_v, rows_a, rows_b, rows_c, rows_d, rows_e, rows_f,
                agg_sh, sem_a, sem_b, sem_c, sem_d, sem_e, sem_f):
    c = lax.axis_index("c")
    s = lax.axis_index("s")

    zeros = jnp.zeros((16,), jnp.float32)

    def zfill(i, carry):
        for k in range(DH // 16):
            rows_a[i, pl.ds(k * 16, 16)] = zeros
        return carry

    lax.fori_loop(0, CH, zfill, 0)

    @pl.when(s < N // STRIPE)
    def _zero():
        for t in range(STRIPE // CH):
            pltpu.sync_copy(rows_a,
                            agg_sh.at[pl.ds(s * STRIPE + t * CH, CH)])
        pltpu.sync_copy(rows_a.at[pl.ds(0, STRIPE % CH)],
                        agg_sh.at[pl.ds(s * STRIPE + STRIPE - STRIPE % CH,
                                        STRIPE % CH)])

    plsc.subcore_barrier()

    pltpu.sync_copy(eg_hbm.at[0, s], src_v)
    pltpu.sync_copy(eg_hbm.at[1, s], dst_v)

    h_half = h_hbm.at[c]
    bufs = (rows_a, rows_b, rows_c, rows_d, rows_e, rows_f)
    sems = (sem_a, sem_b, sem_c, sem_d, sem_e, sem_f)
    depth = NBUF - 1

    def fire_gather(j, u):
        pltpu.async_copy(h_half.at[src_v.at[j]], bufs[u], sems[u])

    def wait_gather(j, u):
        pltpu.make_async_copy(h_half.at[src_v.at[j]], bufs[u],
                              sems[u]).wait()

    for j in range(depth):
        fire_gather(j, j)

    NMAIN = (NCHUNK - 2 * depth) // NBUF

    def body(k, carry):
        j0 = k * NBUF
        for u in range(NBUF):
            j = j0 + u
            wait_gather(j, u)
            fire_gather(j + depth, (u + depth) % NBUF)
            pltpu.sync_copy(bufs[u], agg_sh.at[dst_v.at[j]], add=True)
        return carry

    lax.fori_loop(0, NMAIN, body, 0)

    for j in range(NMAIN * NBUF, NCHUNK):
        u = j % NBUF
        wait_gather(j, u)
        if j + depth < NCHUNK:
            fire_gather(j + depth, (u + depth) % NBUF)
        pltpu.sync_copy(bufs[u], agg_sh.at[dst_v.at[j]], add=True)
    plsc.subcore_barrier()

    @pl.when(s < N // STRIPE)
    def _copy_out():
        pltpu.sync_copy(agg_sh.at[pl.ds(s * STRIPE, STRIPE)],
                        out_hbm.at[c, pl.ds(s * STRIPE, STRIPE)])


def _mm1_body(deg_ref, x_ref, w_ref, h_ref):
    deg = jnp.sum(deg_ref[0], axis=0)
    nsrc = lax.rsqrt(jnp.maximum(deg, 1.0))
    xb = x_ref[...] * nsrc[:, None]
    res = jnp.dot(xb, w_ref[...],
                  preferred_element_type=jnp.float32)
    h_ref[0] = res[:, :DH]
    h_ref[1] = res[:, DH:]


def _mm1(deg_parts, x, w_conv):
    return pl.pallas_call(
        _mm1_body,
        grid=(GRID,),
        in_specs=[
            pl.BlockSpec((1, NW, RB), lambda i: (i, 0, 0)),
            pl.BlockSpec((RB, D), lambda i: (i, 0)),
            pl.BlockSpec((D, D), lambda i: (0, 0)),
        ],
        out_specs=pl.BlockSpec((NC, RB, DH), lambda i: (0, i, 0)),
        out_shape=jax.ShapeDtypeStruct((NC, N, DH), jnp.float32),
    )(deg_parts, x, w_conv)


def _mlp_body(agg_ref, deg_ref, bc_ref, wf_ref, bf_ref, wf2_ref, bf2_ref,
              out_ref):
    agg = jnp.concatenate([agg_ref[0], agg_ref[1]], axis=-1)
    deg = jnp.sum(deg_ref[0], axis=0)
    ndst = lax.rsqrt(jnp.maximum(deg, 1.0))
    h = agg * ndst[:, None] + bc_ref[...]
    h = jnp.maximum(h, 0.0)
    h = jnp.dot(h, wf_ref[...],
                preferred_element_type=jnp.float32) + bf_ref[...]
    h = jnp.maximum(h, 0.0)
    out_ref[...] = jnp.dot(h, wf2_ref[...],
                           preferred_element_type=jnp.float32) + bf2_ref[...]


def _mlp(agg_parts, deg_parts, b_conv, w_fc, b_fc, w_fc2, b_fc2):
    return pl.pallas_call(
        _mlp_body,
        grid=(GRID,),
        in_specs=[
            pl.BlockSpec((NC, RB, DH), lambda i: (0, i, 0)),
            pl.BlockSpec((1, NW, RB), lambda i: (i, 0, 0)),
            pl.BlockSpec((1, D), lambda i: (0, 0)),
            pl.BlockSpec((D, D), lambda i: (0, 0)),
            pl.BlockSpec((1, D), lambda i: (0, 0)),
            pl.BlockSpec((D, D), lambda i: (0, 0)),
            pl.BlockSpec((1, D), lambda i: (0, 0)),
        ],
        out_specs=pl.BlockSpec((RB, D), lambda i: (i, 0)),
        out_shape=jax.ShapeDtypeStruct((N, D), jnp.float32),
    )(agg_parts, deg_parts, b_conv, w_fc, b_fc, w_fc2, b_fc2)


def kernel(x, edge_index, W_conv, b_conv, W_fc, b_fc, W_fc2, b_fc2):
    ei = edge_index.astype(jnp.int32)
    eh = ei.reshape(2, NW, EPT)
    eg = ei.reshape(2, NS, NCHUNK, CH)

    deg_out_parts, deg_in_parts = _deg_kernel(eh)
    h = _mm1(deg_out_parts, x, W_conv)
    agg_parts = _agg_kernel(h, eg)
    return _mlp(agg_parts, deg_in_parts, b_conv.reshape(1, D),
                W_fc, b_fc.reshape(1, D), W_fc2, b_fc2.reshape(1, D))
